# baseline (speedup 1.0000x reference)
import functools

import jax
import jax.numpy as jnp
from jax import lax
from jax.experimental import pallas as pl
from jax.experimental.pallas import tpu as pltpu
from jax.experimental.pallas import tpu_sc as plsc

_N_CLS = 100
_CTX_LEN = 77
_N_CTX = 4
_PREFIX = 4
_EMBED = 512
_LANES = 16
_NW = 32


def sc_full(table, ctx, idx4, idx1):
    mesh = plsc.VectorSubcoreMesh(core_axis_name="c", subcore_axis_name="s")

    @functools.partial(
        pl.kernel,
        out_type=jax.ShapeDtypeStruct((_N_CLS, _CTX_LEN, _EMBED), jnp.float32),
        mesh=mesh,
        scratch_types=[
            pltpu.VMEM((_PREFIX,), jnp.int32),
            pltpu.VMEM((1,), jnp.int32),
            pltpu.VMEM((_PREFIX, _EMBED), jnp.float32),
            pltpu.VMEM((1, _EMBED), jnp.float32),
            pltpu.VMEM((_N_CTX, _EMBED), jnp.float32),
            pltpu.VMEM((1, _CTX_LEN, _EMBED), jnp.float32),
            pltpu.SemaphoreType.DMA,
            pltpu.SemaphoreType.DMA,
        ],
    )
    def body(table_hbm, ctx_hbm, idx4_hbm, idx1_hbm, out_hbm, idx4_v, idx1_v,
             rows4_v, rows1_v, ctx_v, prompt, gsem, ssem):
        cid = lax.axis_index("c")
        sid = lax.axis_index("s")
        wid = sid * 2 + cid

        pltpu.sync_copy(idx4_hbm, idx4_v)
        pltpu.sync_copy(idx1_hbm, idx1_v)
        g1 = pltpu.async_copy(table_hbm.at[idx4_v], rows4_v, gsem)
        g2 = pltpu.async_copy(table_hbm.at[idx1_v], rows1_v, gsem)
        c1 = pltpu.async_copy(ctx_hbm, ctx_v, ssem)
        zeros = jnp.zeros((_LANES,), jnp.float32)
        for r in range(_PREFIX + _N_CTX, _CTX_LEN - 1):
            for k in range(_EMBED // _LANES):
                prompt[0, r, pl.ds(k * _LANES, _LANES)] = zeros
        g1.wait()
        g2.wait()
        c1.wait()
        for r in range(_PREFIX):
            for k in range(_EMBED // _LANES):
                sl = pl.ds(k * _LANES, _LANES)
                prompt[0, r, sl] = rows4_v[r, sl]
        for r in range(_N_CTX):
            for k in range(_EMBED // _LANES):
                sl = pl.ds(k * _LANES, _LANES)
                prompt[0, _PREFIX + r, sl] = ctx_v[r, sl]
        for k in range(_EMBED // _LANES):
            sl = pl.ds(k * _LANES, _LANES)
            prompt[0, _CTX_LEN - 1, sl] = rows1_v[0, sl]

        def cls_body(i, _):
            c = wid + i * _NW

            @pl.when(c < _N_CLS)
            def _():
                pltpu.sync_copy(prompt, out_hbm.at[pl.ds(c, 1)])

            return 0

        lax.fori_loop(0, 4, cls_body, 0)

    return body(table, ctx, idx4, idx1)


def kernel(token_embedding, ctx_vectors, tokenized_prompt):
    idx4 = tokenized_prompt[:_PREFIX]
    idx1 = tokenized_prompt[_CTX_LEN - 1:]
    return sc_full(token_embedding, ctx_vectors, idx4, idx1)


# speedup vs baseline: 4.2488x; 4.2488x over previous
import jax
import jax.numpy as jnp
from jax.experimental import pallas as pl
from jax.experimental.pallas import tpu as pltpu

_N_CLS = 100
_CTX_LEN = 77
_N_CTX = 4
_PREFIX = 4
_EMBED = 512
_ZEROS = _CTX_LEN - _PREFIX - _N_CTX - 1
_CB = 20


def _gather_copies(idx_ref, table_ref, prompt, gsem):
    copies = [
        pltpu.make_async_copy(
            table_ref.at[pl.ds(idx_ref[i], 1)], prompt.at[pl.ds(i, 1)], gsem
        )
        for i in range(_PREFIX)
    ]
    copies.append(
        pltpu.make_async_copy(
            table_ref.at[pl.ds(idx_ref[_PREFIX], 1)],
            prompt.at[pl.ds(_CTX_LEN - 1, 1)],
            gsem,
        )
    )
    return copies


def _tc_full(table, ctx, idx8):
    def body(idx_ref, table_ref, ctx_ref, o_ref, prompt, gsem):
        @pl.when(pl.program_id(0) == 0)
        def _():
            for cp in _gather_copies(idx_ref, table_ref, prompt, gsem):
                cp.start()
            prompt[pl.ds(_PREFIX, _N_CTX), :] = ctx_ref[...]
            prompt[pl.ds(_PREFIX + _N_CTX, _ZEROS), :] = jnp.zeros(
                (_ZEROS, _EMBED), jnp.float32
            )
            for cp in _gather_copies(idx_ref, table_ref, prompt, gsem):
                cp.wait()

        @pl.when(pl.program_id(0) < 2)
        def _write():
            o_ref[...] = jnp.broadcast_to(
                prompt[...][None], (_CB, _CTX_LEN, _EMBED)
            )

    grid_spec = pltpu.PrefetchScalarGridSpec(
        num_scalar_prefetch=1,
        grid=(_N_CLS // _CB,),
        in_specs=[
            pl.BlockSpec(memory_space=pl.ANY),
            pl.BlockSpec((_N_CTX, _EMBED), lambda i, idx: (0, 0)),
        ],
        out_specs=pl.BlockSpec((_CB, _CTX_LEN, _EMBED), lambda i, idx: (i, 0, 0)),
        scratch_shapes=[
            pltpu.VMEM((_CTX_LEN, _EMBED), jnp.float32),
            pltpu.SemaphoreType.DMA,
        ],
    )
    return pl.pallas_call(
        body,
        grid_spec=grid_spec,
        out_shape=jax.ShapeDtypeStruct((_N_CLS, _CTX_LEN, _EMBED), jnp.float32),
    )(idx8, table, ctx)


def kernel(token_embedding, ctx_vectors, tokenized_prompt):
    idx8 = jnp.concatenate(
        [
            tokenized_prompt[:_PREFIX],
            tokenized_prompt[_CTX_LEN - 1 :],
            jnp.zeros((3,), jnp.int32),
        ]
    )
    return _tc_full(token_embedding, ctx_vectors, idx8)


# speedup vs baseline: 4.2944x; 1.0107x over previous
import jax
import jax.numpy as jnp
from jax.experimental import pallas as pl
from jax.experimental.pallas import tpu as pltpu

_N_CLS = 100
_CTX_LEN = 77
_N_CTX = 4
_PREFIX = 4
_EMBED = 512
_ZEROS = _CTX_LEN - _PREFIX - _N_CTX - 1
_CB = 20


def _gather_copies(idx_ref, table_ref, prompt, gsem):
    copies = [
        pltpu.make_async_copy(
            table_ref.at[pl.ds(idx_ref[i], 1)], prompt.at[pl.ds(i, 1)], gsem
        )
        for i in range(_PREFIX)
    ]
    copies.append(
        pltpu.make_async_copy(
            table_ref.at[pl.ds(idx_ref[_PREFIX], 1)],
            prompt.at[pl.ds(_PREFIX + 1, 1)],
            gsem,
        )
    )
    return copies


def _tc_full(table, ctx, idx8):
    def body(idx_ref, table_ref, ctx_ref, o_ref, prompt, gsem):
        i = pl.program_id(0)

        @pl.when(i == 0)
        def _start():
            for cp in _gather_copies(idx_ref, table_ref, prompt, gsem):
                cp.start()

        @pl.when(i < 2)
        def _bulk():
            dense = jnp.concatenate(
                [ctx_ref[...], jnp.zeros((_ZEROS, _EMBED), jnp.float32)],
                axis=0,
            )
            o_ref[:, pl.ds(_PREFIX, _N_CTX + _ZEROS), :] = jnp.broadcast_to(
                dense[None], (_CB, _N_CTX + _ZEROS, _EMBED)
            )

        @pl.when(i == 0)
        def _drain():
            for cp in _gather_copies(idx_ref, table_ref, prompt, gsem):
                cp.wait()

        @pl.when(i < 2)
        def _head():
            o_ref[:, pl.ds(0, _PREFIX), :] = jnp.broadcast_to(
                prompt[pl.ds(0, _PREFIX), :][None], (_CB, _PREFIX, _EMBED)
            )
            o_ref[:, pl.ds(_CTX_LEN - 1, 1), :] = jnp.broadcast_to(
                prompt[pl.ds(_PREFIX + 1, 1), :][None], (_CB, 1, _EMBED)
            )

    grid_spec = pltpu.PrefetchScalarGridSpec(
        num_scalar_prefetch=1,
        grid=(_N_CLS // _CB,),
        in_specs=[
            pl.BlockSpec(memory_space=pl.ANY),
            pl.BlockSpec((_N_CTX, _EMBED), lambda i, idx: (0, 0)),
        ],
        out_specs=pl.BlockSpec((_CB, _CTX_LEN, _EMBED), lambda i, idx: (i, 0, 0)),
        scratch_shapes=[
            pltpu.VMEM((8, _EMBED), jnp.float32),
            pltpu.SemaphoreType.DMA,
        ],
    )
    return pl.pallas_call(
        body,
        grid_spec=grid_spec,
        out_shape=jax.ShapeDtypeStruct((_N_CLS, _CTX_LEN, _EMBED), jnp.float32),
    )(idx8, table, ctx)


def kernel(token_embedding, ctx_vectors, tokenized_prompt):
    idx8 = jnp.concatenate(
        [
            tokenized_prompt[:_PREFIX],
            tokenized_prompt[_CTX_LEN - 1 :],
            jnp.zeros((3,), jnp.int32),
        ]
    )
    return _tc_full(token_embedding, ctx_vectors, idx8)


# speedup vs baseline: 6.0541x; 1.4098x over previous
import jax
import jax.numpy as jnp
from jax.experimental import pallas as pl
from jax.experimental.pallas import tpu as pltpu

_N_CLS = 100
_CTX_LEN = 77
_N_CTX = 4
_PREFIX = 4
_EMBED = 512
_ZEROS = _CTX_LEN - _PREFIX - _N_CTX - 1
_CB = 20


def _gather_copies(idx_ref, table_ref, prompt, gsem):
    copies = [
        pltpu.make_async_copy(
            table_ref.at[pl.ds(idx_ref[i], 1)], prompt.at[pl.ds(i, 1)], gsem
        )
        for i in range(_PREFIX)
    ]
    copies.append(
        pltpu.make_async_copy(
            table_ref.at[pl.ds(idx_ref[_CTX_LEN - 1], 1)],
            prompt.at[pl.ds(_PREFIX + 1, 1)],
            gsem,
        )
    )
    return copies


def _tc_full(table, ctx, tokens):
    def body(idx_ref, table_ref, ctx_ref, o_ref, prompt, gsem):
        i = pl.program_id(0)

        @pl.when(i == 0)
        def _start():
            for cp in _gather_copies(idx_ref, table_ref, prompt, gsem):
                cp.start()

        @pl.when(i < 2)
        def _bulk():
            dense = jnp.concatenate(
                [ctx_ref[...], jnp.zeros((_ZEROS, _EMBED), jnp.float32)],
                axis=0,
            )
            o_ref[:, pl.ds(_PREFIX, _N_CTX + _ZEROS), :] = jnp.broadcast_to(
                dense[None], (_CB, _N_CTX + _ZEROS, _EMBED)
            )

        @pl.when(i == 0)
        def _drain():
            for cp in _gather_copies(idx_ref, table_ref, prompt, gsem):
                cp.wait()

        @pl.when(i < 2)
        def _head():
            o_ref[:, pl.ds(0, _PREFIX), :] = jnp.broadcast_to(
                prompt[pl.ds(0, _PREFIX), :][None], (_CB, _PREFIX, _EMBED)
            )
            o_ref[:, pl.ds(_CTX_LEN - 1, 1), :] = jnp.broadcast_to(
                prompt[pl.ds(_PREFIX + 1, 1), :][None], (_CB, 1, _EMBED)
            )

    grid_spec = pltpu.PrefetchScalarGridSpec(
        num_scalar_prefetch=1,
        grid=(_N_CLS // _CB,),
        in_specs=[
            pl.BlockSpec(memory_space=pl.ANY),
            pl.BlockSpec((_N_CTX, _EMBED), lambda i, idx: (0, 0)),
        ],
        out_specs=pl.BlockSpec((_CB, _CTX_LEN, _EMBED), lambda i, idx: (i, 0, 0)),
        scratch_shapes=[
            pltpu.VMEM((8, _EMBED), jnp.float32),
            pltpu.SemaphoreType.DMA,
        ],
    )
    return pl.pallas_call(
        body,
        grid_spec=grid_spec,
        out_shape=jax.ShapeDtypeStruct((_N_CLS, _CTX_LEN, _EMBED), jnp.float32),
    )(tokens, table, ctx)


def kernel(token_embedding, ctx_vectors, tokenized_prompt):
    return _tc_full(token_embedding, ctx_vectors, tokenized_prompt)


# speedup vs baseline: 6.0617x; 1.0013x over previous
import jax
import jax.numpy as jnp
from jax.experimental import pallas as pl
from jax.experimental.pallas import tpu as pltpu

_N_CLS = 100
_CTX_LEN = 77
_N_CTX = 4
_PREFIX = 4
_EMBED = 512
_ZEROS = _CTX_LEN - _PREFIX - _N_CTX - 1
_CB = 25


def _gather_copies(idx_ref, table_ref, prompt, gsem):
    copies = [
        pltpu.make_async_copy(
            table_ref.at[pl.ds(idx_ref[i], 1)], prompt.at[pl.ds(i, 1)], gsem
        )
        for i in range(_PREFIX)
    ]
    copies.append(
        pltpu.make_async_copy(
            table_ref.at[pl.ds(idx_ref[_CTX_LEN - 1], 1)],
            prompt.at[pl.ds(_PREFIX + 1, 1)],
            gsem,
        )
    )
    return copies


def _tc_full(table, ctx, tokens):
    def body(idx_ref, table_ref, ctx_ref, o_ref, prompt, gsem):
        i = pl.program_id(0)

        @pl.when(i == 0)
        def _start():
            for cp in _gather_copies(idx_ref, table_ref, prompt, gsem):
                cp.start()

        @pl.when(i < 2)
        def _bulk():
            dense = jnp.concatenate(
                [ctx_ref[...], jnp.zeros((_ZEROS, _EMBED), jnp.float32)],
                axis=0,
            )
            o_ref[:, pl.ds(_PREFIX, _N_CTX + _ZEROS), :] = jnp.broadcast_to(
                dense[None], (_CB, _N_CTX + _ZEROS, _EMBED)
            )

        @pl.when(i == 0)
        def _drain():
            for cp in _gather_copies(idx_ref, table_ref, prompt, gsem):
                cp.wait()

        @pl.when(i < 2)
        def _head():
            o_ref[:, pl.ds(0, _PREFIX), :] = jnp.broadcast_to(
                prompt[pl.ds(0, _PREFIX), :][None], (_CB, _PREFIX, _EMBED)
            )
            o_ref[:, pl.ds(_CTX_LEN - 1, 1), :] = jnp.broadcast_to(
                prompt[pl.ds(_PREFIX + 1, 1), :][None], (_CB, 1, _EMBED)
            )

    grid_spec = pltpu.PrefetchScalarGridSpec(
        num_scalar_prefetch=1,
        grid=(_N_CLS // _CB,),
        in_specs=[
            pl.BlockSpec(memory_space=pl.ANY),
            pl.BlockSpec((_N_CTX, _EMBED), lambda i, idx: (0, 0)),
        ],
        out_specs=pl.BlockSpec((_CB, _CTX_LEN, _EMBED), lambda i, idx: (i, 0, 0)),
        scratch_shapes=[
            pltpu.VMEM((8, _EMBED), jnp.float32),
            pltpu.SemaphoreType.DMA,
        ],
    )
    return pl.pallas_call(
        body,
        grid_spec=grid_spec,
        out_shape=jax.ShapeDtypeStruct((_N_CLS, _CTX_LEN, _EMBED), jnp.float32),
    )(tokens, table, ctx)


def kernel(token_embedding, ctx_vectors, tokenized_prompt):
    return _tc_full(token_embedding, ctx_vectors, tokenized_prompt)
